# baseline (speedup 1.0000x reference)
import functools

import jax
import jax.numpy as jnp
from jax import lax
from jax.experimental import pallas as pl
from jax.experimental.pallas import tpu as pltpu
from jax.experimental.pallas import tpu_sc as plsc

VAR_IDX = 3
D = 32
NC = 2
NS = 16
NW = NC * NS
SUB = 128
SPC = 2
CHUNK = SUB * SPC
L = 16
VCHUNK = 512
QCHUNK = VCHUNK // 4


def _make_pack(V):
    n_full = V // VCHUNK
    tail_v = V - n_full * VCHUNK
    mesh = plsc.VectorSubcoreMesh(core_axis_name="c", subcore_axis_name="s")

    @functools.partial(
        pl.kernel,
        mesh=mesh,
        out_type=jax.ShapeDtypeStruct((V * D // 128, 128), jnp.float32),
        scratch_types=[
            pltpu.VMEM((16, 8, 128), jnp.float32),
            pltpu.VMEM((16, 8, 128), jnp.float32),
            pltpu.VMEM((QCHUNK, 128), jnp.float32),
            pltpu.VMEM((QCHUNK, 128), jnp.float32),
            pltpu.SemaphoreType.DMA,
            pltpu.SemaphoreType.DMA,
            pltpu.SemaphoreType.DMA,
            pltpu.SemaphoreType.DMA,
        ],
        compiler_params=pltpu.CompilerParams(
            use_tc_tiling_on_sc=True,
            needs_layout_passes=False, disable_bounds_checks=True),
    )
    def body(tablet_hbm, tail_hbm, tq_hbm, in0, in1, o0, o1,
             isem0, isem1, osem0, osem1):
        wid = lax.axis_index("s") * NC + lax.axis_index("c")
        iota16 = lax.iota(jnp.int32, L)
        iota4q = iota16 // 4
        colb = (iota16 % 4) * D

        def fire_in(c, inb, isem):
            for ti in range(16):
                pltpu.async_copy(
                    tablet_hbm.at[pl.ds(8 * (ti // 4), 8),
                                  pl.ds(c * VCHUNK + 128 * (ti % 4), 128)],
                    inb.at[ti], isem)

        def wait_in(inb, isem):
            for ti in range(16):
                pltpu.make_async_copy(
                    tablet_hbm.at[pl.ds(0, 8), pl.ds(0, 128)],
                    inb.at[ti], isem).wait()

        def shuffle(inb, ob):
            def sh_body(p, carry):
                jv = p // 8
                g16 = p % 8
                rows = iota4q + jv * 32 + g16 * 4
                for d in range(D):
                    v = inb[(d // 8) * 4 + jv, d % 8, pl.ds(g16 * L, L)]
                    plsc.store_scatter(ob, [rows, colb + d], v)
                return carry
            lax.fori_loop(0, 32, sh_body, 0)

        def fire_out(c, ob, osem):
            pltpu.async_copy(tq_hbm.at[pl.ds(c * QCHUNK, QCHUNK)], ob, osem)

        def wait_out(ob, osem):
            pltpu.make_async_copy(
                ob, tq_hbm.at[pl.ds(0, QCHUNK)], osem).wait()

        n_mine = (n_full - wid + NW - 1) // NW

        fire_in(wid, in0, isem0)

        def pair(j, carry):
            c0 = wid + 2 * j * NW
            c1 = c0 + NW

            @pl.when(2 * j + 1 < n_mine)
            def _():
                fire_in(c1, in1, isem1)
            wait_in(in0, isem0)

            @pl.when(j > 0)
            def _():
                wait_out(o0, osem0)
            shuffle(in0, o0)
            pltpu.async_copy(o0, tq_hbm.at[pl.ds(c0 * QCHUNK, QCHUNK)], osem0)

            @pl.when(2 * j + 1 < n_mine)
            def _():
                c2 = c0 + 2 * NW

                @pl.when(2 * j + 2 < n_mine)
                def _():
                    fire_in(c2, in0, isem0)
                wait_in(in1, isem1)

                @pl.when(j > 0)
                def _():
                    wait_out(o1, osem1)
                shuffle(in1, o1)
                pltpu.async_copy(
                    o1, tq_hbm.at[pl.ds(c1 * QCHUNK, QCHUNK)], osem1)
            return carry

        lax.fori_loop(0, (n_mine + 1) // 2, pair, 0)

        @pl.when(n_mine > 0)
        def _():
            wait_out(o0, osem0)

        @pl.when(n_mine > 1)
        def _():
            wait_out(o1, osem1)

        if tail_v:
            @pl.when(wid == NW - 1)
            def _():
                tq0 = (n_full * VCHUNK) // 4
                tq_n = tail_v // 4
                pltpu.sync_copy(tail_hbm, o0.at[pl.ds(0, tq_n)])
                pltpu.sync_copy(o0.at[pl.ds(0, tq_n)],
                                tq_hbm.at[pl.ds(tq0, tq_n)])

    return body


def _make_gather(S, Bt):
    B = S * Bt
    b_per_w = B // NW
    n_idx_rows = b_per_w // SUB
    cpw = b_per_w // CHUNK
    n_pairs = cpw // 2
    q_per_s = Bt // CHUNK
    rows_per_chunk = CHUNK // SUB
    mesh = plsc.VectorSubcoreMesh(core_axis_name="c", subcore_axis_name="s")

    @functools.partial(
        pl.kernel,
        mesh=mesh,
        out_type=jax.ShapeDtypeStruct((S, D, Bt), jnp.float32),
        scratch_types=[
            pltpu.VMEM((n_idx_rows, SUB), jnp.int32),
            pltpu.VMEM((rows_per_chunk, SUB), jnp.int32),
            pltpu.VMEM((rows_per_chunk, SUB), jnp.int32),
            pltpu.VMEM((CHUNK, SUB), jnp.float32),
            pltpu.VMEM((CHUNK, SUB), jnp.float32),
            pltpu.VMEM((D, CHUNK), jnp.float32),
            pltpu.VMEM((D, CHUNK), jnp.float32),
            pltpu.SemaphoreType.DMA,
            pltpu.SemaphoreType.DMA,
            pltpu.SemaphoreType.DMA,
            pltpu.SemaphoreType.DMA,
        ],
        compiler_params=pltpu.CompilerParams(
            use_tc_tiling_on_sc=False, needs_layout_passes=False,
            disable_bounds_checks=True),
    )
    def body(idx_hbm, tableq_hbm, out_hbm, idx_v, iq0, iq1, g0, g1, t0, t1,
             gsem0, gsem1, osem0, osem1):
        wid = lax.axis_index("s") * NC + lax.axis_index("c")
        h0 = wid * cpw
        pltpu.sync_copy(idx_hbm.at[wid], idx_v)
        iota16 = lax.iota(jnp.int32, L)

        def fire_g(hl, iq, g, gsem):
            for r in range(rows_per_chunk):
                for c in range(SUB // L):
                    iq[r, pl.ds(c * L, L)] = lax.shift_right_logical(
                        idx_v[hl * rows_per_chunk + r, pl.ds(c * L, L)], 2)
            for k in range(SPC):
                pltpu.async_copy(
                    tableq_hbm.at[iq.at[k]],
                    g.at[pl.ds(k * SUB, SUB)],
                    gsem,
                )

        def wait_g(g, gsem):
            pltpu.make_async_copy(
                tableq_hbm.at[pl.ds(0, CHUNK)], g, gsem).wait()

        def transpose(hl, g, t):
            def tr_body(gi, carry):
                rows = gi * L + iota16
                idx16 = idx_v[hl * rows_per_chunk + gi // (SUB // L),
                              pl.ds((gi % (SUB // L)) * L, L)]
                m32 = lax.shift_left(jnp.bitwise_and(idx16, 3), 5)
                for d in range(D):
                    t[d, pl.ds(gi * L, L)] = plsc.load_gather(
                        g, [rows, m32 + d])
                return carry
            lax.fori_loop(0, CHUNK // L, tr_body, 0)

        def fire_w(hl, t, osem):
            h = h0 + hl
            s = h // q_per_s
            b0 = (h % q_per_s) * CHUNK
            pltpu.async_copy(t, out_hbm.at[s, :, pl.ds(b0, CHUNK)], osem)

        def wait_w(t, osem):
            pltpu.make_async_copy(
                t, out_hbm.at[0, :, pl.ds(0, CHUNK)], osem).wait()

        fire_g(0, iq0, g0, gsem0)

        def pair(j, carry):
            hl = 2 * j
            wait_g(g0, gsem0)
            fire_g(hl + 1, iq1, g1, gsem1)

            @pl.when(j > 0)
            def _():
                wait_w(t0, osem0)
            transpose(hl, g0, t0)
            fire_w(hl, t0, osem0)

            wait_g(g1, gsem1)

            @pl.when(j < n_pairs - 1)
            def _():
                fire_g(hl + 2, iq0, g0, gsem0)

            @pl.when(j > 0)
            def _():
                wait_w(t1, osem1)
            transpose(hl + 1, g1, t1)
            fire_w(hl + 1, t1, osem1)
            return carry

        lax.fori_loop(0, n_pairs, pair, 0)
        wait_w(t0, osem0)
        wait_w(t1, osem1)

    return body


def kernel(X, table):
    Bt, S, _ = X.shape
    V = table.shape[0]
    idx3 = jnp.transpose(X, (2, 1, 0))[VAR_IDX].reshape(NW, S * Bt // (NW * SUB), SUB)
    tablet = jnp.transpose(table)
    n_tail = V % VCHUNK
    tail16 = table[V - n_tail:].reshape(n_tail // 4, 128)
    tableq = _make_pack(V)(tablet, tail16)
    out3 = _make_gather(S, Bt)(idx3, tableq)
    return jnp.transpose(out3, (2, 0, 1))


# speedup vs baseline: 1.3995x; 1.3995x over previous
import functools

import jax
import jax.numpy as jnp
from jax import lax
from jax.experimental import pallas as pl
from jax.experimental.pallas import tpu as pltpu
from jax.experimental.pallas import tpu_sc as plsc

VAR_IDX = 3
D = 32
NC = 2
NS = 16
NW = NC * NS
SUB = 128
SPC = 2
CHUNK = SUB * SPC
L = 16
VCHUNK = 512
QCHUNK = VCHUNK // 4


def _make_pack(V):
    n_full = V // VCHUNK
    tail_v = V - n_full * VCHUNK
    mesh = plsc.VectorSubcoreMesh(core_axis_name="c", subcore_axis_name="s")

    @functools.partial(
        pl.kernel,
        mesh=mesh,
        out_type=jax.ShapeDtypeStruct((V * D // 128, 128), jnp.float32),
        scratch_types=[
            pltpu.VMEM((16, 8, 128), jnp.float32),
            pltpu.VMEM((16, 8, 128), jnp.float32),
            pltpu.VMEM((QCHUNK, 128), jnp.float32),
            pltpu.VMEM((QCHUNK, 128), jnp.float32),
            pltpu.SemaphoreType.DMA,
            pltpu.SemaphoreType.DMA,
            pltpu.SemaphoreType.DMA,
            pltpu.SemaphoreType.DMA,
        ],
        compiler_params=pltpu.CompilerParams(
            use_tc_tiling_on_sc=True,
            needs_layout_passes=False, disable_bounds_checks=True),
    )
    def body(tablet_hbm, tail_hbm, tq_hbm, in0, in1, o0, o1,
             isem0, isem1, osem0, osem1):
        wid = lax.axis_index("s") * NC + lax.axis_index("c")
        iota16 = lax.iota(jnp.int32, L)
        iota4q = iota16 // 4
        colb = (iota16 % 4) * D

        def fire_in(c, inb, isem):
            for ti in range(16):
                pltpu.async_copy(
                    tablet_hbm.at[pl.ds(8 * (ti // 4), 8),
                                  pl.ds(c * VCHUNK + 128 * (ti % 4), 128)],
                    inb.at[ti], isem)

        def wait_in(inb, isem):
            for ti in range(16):
                pltpu.make_async_copy(
                    tablet_hbm.at[pl.ds(0, 8), pl.ds(0, 128)],
                    inb.at[ti], isem).wait()

        def shuffle(inb, ob):
            def sh_body(p, carry):
                jv = p // 8
                g16 = p % 8
                rows = iota4q + jv * 32 + g16 * 4
                for d0 in range(0, D, 8):
                    vs = [inb[(d // 8) * 4 + jv, d % 8, pl.ds(g16 * L, L)]
                          for d in range(d0, d0 + 8)]
                    for i, d in enumerate(range(d0, d0 + 8)):
                        plsc.store_scatter(ob, [rows, colb + d], vs[i])
                return carry
            lax.fori_loop(0, 32, sh_body, 0)

        def fire_out(c, ob, osem):
            pltpu.async_copy(tq_hbm.at[pl.ds(c * QCHUNK, QCHUNK)], ob, osem)

        def wait_out(ob, osem):
            pltpu.make_async_copy(
                ob, tq_hbm.at[pl.ds(0, QCHUNK)], osem).wait()

        n_mine = (n_full - wid + NW - 1) // NW

        fire_in(wid, in0, isem0)

        def pair(j, carry):
            c0 = wid + 2 * j * NW
            c1 = c0 + NW

            @pl.when(2 * j + 1 < n_mine)
            def _():
                fire_in(c1, in1, isem1)
            wait_in(in0, isem0)

            @pl.when(j > 0)
            def _():
                wait_out(o0, osem0)
            shuffle(in0, o0)
            pltpu.async_copy(o0, tq_hbm.at[pl.ds(c0 * QCHUNK, QCHUNK)], osem0)

            @pl.when(2 * j + 1 < n_mine)
            def _():
                c2 = c0 + 2 * NW

                @pl.when(2 * j + 2 < n_mine)
                def _():
                    fire_in(c2, in0, isem0)
                wait_in(in1, isem1)

                @pl.when(j > 0)
                def _():
                    wait_out(o1, osem1)
                shuffle(in1, o1)
                pltpu.async_copy(
                    o1, tq_hbm.at[pl.ds(c1 * QCHUNK, QCHUNK)], osem1)
            return carry

        lax.fori_loop(0, (n_mine + 1) // 2, pair, 0)

        @pl.when(n_mine > 0)
        def _():
            wait_out(o0, osem0)

        @pl.when(n_mine > 1)
        def _():
            wait_out(o1, osem1)

        if tail_v:
            @pl.when(wid == NW - 1)
            def _():
                tq0 = (n_full * VCHUNK) // 4
                tq_n = tail_v // 4
                pltpu.sync_copy(tail_hbm, o0.at[pl.ds(0, tq_n)])
                pltpu.sync_copy(o0.at[pl.ds(0, tq_n)],
                                tq_hbm.at[pl.ds(tq0, tq_n)])

    return body


def _make_gather(S, Bt):
    B = S * Bt
    b_per_w = B // NW
    n_idx_rows = b_per_w // SUB
    cpw = b_per_w // CHUNK
    n_pairs = cpw // 2
    q_per_s = Bt // CHUNK
    rows_per_chunk = CHUNK // SUB
    mesh = plsc.VectorSubcoreMesh(core_axis_name="c", subcore_axis_name="s")

    @functools.partial(
        pl.kernel,
        mesh=mesh,
        out_type=jax.ShapeDtypeStruct((S, D, Bt), jnp.float32),
        scratch_types=[
            pltpu.VMEM((n_idx_rows, SUB), jnp.int32),
            pltpu.VMEM((rows_per_chunk, SUB), jnp.int32),
            pltpu.VMEM((rows_per_chunk, SUB), jnp.int32),
            pltpu.VMEM((CHUNK, SUB), jnp.float32),
            pltpu.VMEM((CHUNK, SUB), jnp.float32),
            pltpu.VMEM((D, CHUNK), jnp.float32),
            pltpu.VMEM((D, CHUNK), jnp.float32),
            pltpu.SemaphoreType.DMA,
            pltpu.SemaphoreType.DMA,
            pltpu.SemaphoreType.DMA,
            pltpu.SemaphoreType.DMA,
        ],
        compiler_params=pltpu.CompilerParams(
            use_tc_tiling_on_sc=False, needs_layout_passes=False,
            disable_bounds_checks=True),
    )
    def body(idx_hbm, tableq_hbm, out_hbm, idx_v, iq0, iq1, g0, g1, t0, t1,
             gsem0, gsem1, osem0, osem1):
        wid = lax.axis_index("s") * NC + lax.axis_index("c")
        h0 = wid * cpw
        pltpu.sync_copy(idx_hbm.at[wid], idx_v)
        iota16 = lax.iota(jnp.int32, L)

        def fire_g(hl, iq, g, gsem):
            for r in range(rows_per_chunk):
                for c in range(SUB // L):
                    iq[r, pl.ds(c * L, L)] = lax.shift_right_logical(
                        idx_v[hl * rows_per_chunk + r, pl.ds(c * L, L)], 2)
            for k in range(SPC):
                pltpu.async_copy(
                    tableq_hbm.at[iq.at[k]],
                    g.at[pl.ds(k * SUB, SUB)],
                    gsem,
                )

        def wait_g(g, gsem):
            pltpu.make_async_copy(
                tableq_hbm.at[pl.ds(0, CHUNK)], g, gsem).wait()

        def transpose(hl, g, t):
            def tr_body(gi, carry):
                rows = gi * L + iota16
                idx16 = idx_v[hl * rows_per_chunk + gi // (SUB // L),
                              pl.ds((gi % (SUB // L)) * L, L)]
                m32 = lax.shift_left(jnp.bitwise_and(idx16, 3), 5)
                for d0 in range(0, D, 8):
                    vs = [plsc.load_gather(g, [rows, m32 + d])
                          for d in range(d0, d0 + 8)]
                    for i, d in enumerate(range(d0, d0 + 8)):
                        t[d, pl.ds(gi * L, L)] = vs[i]
                return carry
            lax.fori_loop(0, CHUNK // L, tr_body, 0)

        def fire_w(hl, t, osem):
            h = h0 + hl
            s = h // q_per_s
            b0 = (h % q_per_s) * CHUNK
            pltpu.async_copy(t, out_hbm.at[s, :, pl.ds(b0, CHUNK)], osem)

        def wait_w(t, osem):
            pltpu.make_async_copy(
                t, out_hbm.at[0, :, pl.ds(0, CHUNK)], osem).wait()

        fire_g(0, iq0, g0, gsem0)

        def pair(j, carry):
            hl = 2 * j
            wait_g(g0, gsem0)
            fire_g(hl + 1, iq1, g1, gsem1)

            @pl.when(j > 0)
            def _():
                wait_w(t0, osem0)
            transpose(hl, g0, t0)
            fire_w(hl, t0, osem0)

            wait_g(g1, gsem1)

            @pl.when(j < n_pairs - 1)
            def _():
                fire_g(hl + 2, iq0, g0, gsem0)

            @pl.when(j > 0)
            def _():
                wait_w(t1, osem1)
            transpose(hl + 1, g1, t1)
            fire_w(hl + 1, t1, osem1)
            return carry

        lax.fori_loop(0, n_pairs, pair, 0)
        wait_w(t0, osem0)
        wait_w(t1, osem1)

    return body


def kernel(X, table):
    Bt, S, _ = X.shape
    V = table.shape[0]
    idx3 = jnp.transpose(X, (2, 1, 0))[VAR_IDX].reshape(NW, S * Bt // (NW * SUB), SUB)
    tablet = jnp.transpose(table)
    n_tail = V % VCHUNK
    tail16 = table[V - n_tail:].reshape(n_tail // 4, 128)
    tableq = _make_pack(V)(tablet, tail16)
    out3 = _make_gather(S, Bt)(idx3, tableq)
    return jnp.transpose(out3, (2, 0, 1))


# speedup vs baseline: 1.4240x; 1.0175x over previous
import functools

import jax
import jax.numpy as jnp
from jax import lax
from jax.experimental import pallas as pl
from jax.experimental.pallas import tpu as pltpu
from jax.experimental.pallas import tpu_sc as plsc

VAR_IDX = 3
D = 32
NC = 2
NS = 16
NW = NC * NS
SUB = 128
SPC = 2
CHUNK = SUB * SPC
L = 16
VCHUNK = 512
QCHUNK = VCHUNK // 4


def _make_pack(V):
    n_full = V // VCHUNK
    tail_v = V - n_full * VCHUNK
    mesh = plsc.VectorSubcoreMesh(core_axis_name="c", subcore_axis_name="s")

    @functools.partial(
        pl.kernel,
        mesh=mesh,
        out_type=jax.ShapeDtypeStruct((V * D // 128, 128), jnp.float32),
        scratch_types=[
            pltpu.VMEM((16, 8, 128), jnp.float32),
            pltpu.VMEM((16, 8, 128), jnp.float32),
            pltpu.VMEM((QCHUNK, 128), jnp.float32),
            pltpu.VMEM((QCHUNK, 128), jnp.float32),
            pltpu.SemaphoreType.DMA,
            pltpu.SemaphoreType.DMA,
            pltpu.SemaphoreType.DMA,
            pltpu.SemaphoreType.DMA,
        ],
        compiler_params=pltpu.CompilerParams(
            use_tc_tiling_on_sc=True,
            needs_layout_passes=False, disable_bounds_checks=True),
    )
    def body(tablet_hbm, tail_hbm, tq_hbm, in0, in1, o0, o1,
             isem0, isem1, osem0, osem1):
        wid = lax.axis_index("s") * NC + lax.axis_index("c")
        iota16 = lax.iota(jnp.int32, L)
        iota4q = iota16 // 4
        colb = (iota16 % 4) * D

        def fire_in(c, inb, isem):
            for ti in range(16):
                pltpu.async_copy(
                    tablet_hbm.at[pl.ds(8 * (ti // 4), 8),
                                  pl.ds(c * VCHUNK + 128 * (ti % 4), 128)],
                    inb.at[ti], isem)

        def wait_in(inb, isem):
            for ti in range(16):
                pltpu.make_async_copy(
                    tablet_hbm.at[pl.ds(0, 8), pl.ds(0, 128)],
                    inb.at[ti], isem).wait()

        def shuffle(inb, ob):
            def sh_body(p, carry):
                jv = p // 8
                g16 = p % 8
                rows = iota4q + jv * 32 + g16 * 4
                for d0 in range(0, D, 8):
                    vs = [inb[(d // 8) * 4 + jv, d % 8, pl.ds(g16 * L, L)]
                          for d in range(d0, d0 + 8)]
                    for i, d in enumerate(range(d0, d0 + 8)):
                        plsc.store_scatter(ob, [rows, colb + d], vs[i])
                return carry
            lax.fori_loop(0, 32, sh_body, 0)

        def fire_out(c, ob, osem):
            pltpu.async_copy(tq_hbm.at[pl.ds(c * QCHUNK, QCHUNK)], ob, osem)

        def wait_out(ob, osem):
            pltpu.make_async_copy(
                ob, tq_hbm.at[pl.ds(0, QCHUNK)], osem).wait()

        n_mine = (n_full - wid + NW - 1) // NW

        fire_in(wid, in0, isem0)

        def pair(j, carry):
            c0 = wid + 2 * j * NW
            c1 = c0 + NW

            @pl.when(2 * j + 1 < n_mine)
            def _():
                fire_in(c1, in1, isem1)
            wait_in(in0, isem0)

            @pl.when(j > 0)
            def _():
                wait_out(o0, osem0)
            shuffle(in0, o0)
            pltpu.async_copy(o0, tq_hbm.at[pl.ds(c0 * QCHUNK, QCHUNK)], osem0)

            @pl.when(2 * j + 1 < n_mine)
            def _():
                c2 = c0 + 2 * NW

                @pl.when(2 * j + 2 < n_mine)
                def _():
                    fire_in(c2, in0, isem0)
                wait_in(in1, isem1)

                @pl.when(j > 0)
                def _():
                    wait_out(o1, osem1)
                shuffle(in1, o1)
                pltpu.async_copy(
                    o1, tq_hbm.at[pl.ds(c1 * QCHUNK, QCHUNK)], osem1)
            return carry

        lax.fori_loop(0, (n_mine + 1) // 2, pair, 0)

        @pl.when(n_mine > 0)
        def _():
            wait_out(o0, osem0)

        @pl.when(n_mine > 1)
        def _():
            wait_out(o1, osem1)

        if tail_v:
            @pl.when(wid == NW - 1)
            def _():
                tq0 = (n_full * VCHUNK) // 4
                tq_n = tail_v // 4
                pltpu.sync_copy(tail_hbm, o0.at[pl.ds(0, tq_n)])
                pltpu.sync_copy(o0.at[pl.ds(0, tq_n)],
                                tq_hbm.at[pl.ds(tq0, tq_n)])

    return body


def _make_gather(S, Bt):
    B = S * Bt
    b_per_w = B // NW
    n_idx_rows = b_per_w // SUB
    cpw = b_per_w // CHUNK
    n_pairs = cpw // 2
    q_per_s = Bt // CHUNK
    rows_per_chunk = CHUNK // SUB
    mesh = plsc.VectorSubcoreMesh(core_axis_name="c", subcore_axis_name="s")

    @functools.partial(
        pl.kernel,
        mesh=mesh,
        out_type=jax.ShapeDtypeStruct((S, D, Bt), jnp.float32),
        scratch_types=[
            pltpu.VMEM((n_idx_rows, SUB), jnp.int32),
            pltpu.VMEM((CHUNK, D), jnp.float32),
            pltpu.VMEM((CHUNK, D), jnp.float32),
            pltpu.VMEM((D, CHUNK), jnp.float32),
            pltpu.VMEM((D, CHUNK), jnp.float32),
            pltpu.SemaphoreType.DMA,
            pltpu.SemaphoreType.DMA,
            pltpu.SemaphoreType.DMA,
            pltpu.SemaphoreType.DMA,
        ],
        compiler_params=pltpu.CompilerParams(
            use_tc_tiling_on_sc=False, needs_layout_passes=False,
            disable_bounds_checks=True),
    )
    def body(idx_hbm, table_hbm, out_hbm, idx_v, g0, g1, t0, t1,
             gsem0, gsem1, osem0, osem1):
        wid = lax.axis_index("s") * NC + lax.axis_index("c")
        h0 = wid * cpw
        pltpu.sync_copy(idx_hbm.at[wid], idx_v)
        iota16 = lax.iota(jnp.int32, L)
        cols_d = [jnp.full((L,), d, jnp.int32) for d in range(D)]

        def fire_g(hl, g, gsem):
            for k in range(SPC):
                pltpu.async_copy(
                    table_hbm.at[idx_v.at[hl * rows_per_chunk + k]],
                    g.at[pl.ds(k * SUB, SUB)],
                    gsem,
                )

        def wait_g(g, gsem):
            pltpu.make_async_copy(
                table_hbm.at[pl.ds(0, CHUNK)], g, gsem).wait()

        def transpose(hl, g, t):
            def tr_body(gi, carry):
                rows = gi * L + iota16
                for d0 in range(0, D, 8):
                    vs = [plsc.load_gather(g, [rows, cols_d[d]])
                          for d in range(d0, d0 + 8)]
                    for i, d in enumerate(range(d0, d0 + 8)):
                        t[d, pl.ds(gi * L, L)] = vs[i]
                return carry
            lax.fori_loop(0, CHUNK // L, tr_body, 0)

        def fire_w(hl, t, osem):
            h = h0 + hl
            s = h // q_per_s
            b0 = (h % q_per_s) * CHUNK
            pltpu.async_copy(t, out_hbm.at[s, :, pl.ds(b0, CHUNK)], osem)

        def wait_w(t, osem):
            pltpu.make_async_copy(
                t, out_hbm.at[0, :, pl.ds(0, CHUNK)], osem).wait()

        fire_g(0, g0, gsem0)

        def pair(j, carry):
            hl = 2 * j
            wait_g(g0, gsem0)
            fire_g(hl + 1, g1, gsem1)

            @pl.when(j > 0)
            def _():
                wait_w(t0, osem0)
            transpose(hl, g0, t0)
            fire_w(hl, t0, osem0)

            wait_g(g1, gsem1)

            @pl.when(j < n_pairs - 1)
            def _():
                fire_g(hl + 2, g0, gsem0)

            @pl.when(j > 0)
            def _():
                wait_w(t1, osem1)
            transpose(hl + 1, g1, t1)
            fire_w(hl + 1, t1, osem1)
            return carry

        lax.fori_loop(0, n_pairs, pair, 0)
        wait_w(t0, osem0)
        wait_w(t1, osem1)

    return body


def kernel(X, table):
    Bt, S, _ = X.shape
    V = table.shape[0]
    idx3 = jnp.transpose(X, (2, 1, 0))[VAR_IDX].reshape(NW, S * Bt // (NW * SUB), SUB)
    tablet = jnp.transpose(table)
    n_tail = V % VCHUNK
    tail16 = table[V - n_tail:].reshape(n_tail // 4, 128)
    tableq = _make_pack(V)(tablet, tail16)
    out3 = _make_gather(S, Bt)(idx3, tableq.reshape(V, D))
    return jnp.transpose(out3, (2, 0, 1))


# speedup vs baseline: 1.4362x; 1.0086x over previous
import functools

import jax
import jax.numpy as jnp
from jax import lax
from jax.experimental import pallas as pl
from jax.experimental.pallas import tpu as pltpu
from jax.experimental.pallas import tpu_sc as plsc

VAR_IDX = 3
D = 32
NC = 2
NS = 16
NW = NC * NS
SUB = 128
SPC = 2
CHUNK = SUB * SPC
L = 16
VCHUNK = 512
QCHUNK = VCHUNK // 4


def _make_pack(V):
    n_full = V // VCHUNK
    tail_v = V - n_full * VCHUNK
    mesh = plsc.VectorSubcoreMesh(core_axis_name="c", subcore_axis_name="s")

    @functools.partial(
        pl.kernel,
        mesh=mesh,
        out_type=jax.ShapeDtypeStruct((V * D // 128, 128), jnp.float32),
        scratch_types=[
            pltpu.VMEM((16, 8, 128), jnp.float32),
            pltpu.VMEM((16, 8, 128), jnp.float32),
            pltpu.VMEM((QCHUNK, 128), jnp.float32),
            pltpu.VMEM((QCHUNK, 128), jnp.float32),
            pltpu.SemaphoreType.DMA,
            pltpu.SemaphoreType.DMA,
            pltpu.SemaphoreType.DMA,
            pltpu.SemaphoreType.DMA,
        ],
        compiler_params=pltpu.CompilerParams(
            use_tc_tiling_on_sc=True,
            needs_layout_passes=False, disable_bounds_checks=True),
    )
    def body(tablet_hbm, tail_hbm, tq_hbm, in0, in1, o0, o1,
             isem0, isem1, osem0, osem1):
        wid = lax.axis_index("s") * NC + lax.axis_index("c")
        iota16 = lax.iota(jnp.int32, L)
        iota4q = iota16 // 4
        colb = (iota16 % 4) * D

        def fire_in(c, inb, isem):
            for ti in range(16):
                pltpu.async_copy(
                    tablet_hbm.at[pl.ds(8 * (ti // 4), 8),
                                  pl.ds(c * VCHUNK + 128 * (ti % 4), 128)],
                    inb.at[ti], isem)

        def wait_in(inb, isem):
            for ti in range(16):
                pltpu.make_async_copy(
                    tablet_hbm.at[pl.ds(0, 8), pl.ds(0, 128)],
                    inb.at[ti], isem).wait()

        def shuffle(inb, ob):
            def sh_body(p, carry):
                jv = p // 8
                g16 = p % 8
                rows = iota4q + jv * 32 + g16 * 4
                for d0 in range(0, D, 16):
                    vs = [inb[(d // 8) * 4 + jv, d % 8, pl.ds(g16 * L, L)]
                          for d in range(d0, d0 + 16)]
                    for i, d in enumerate(range(d0, d0 + 16)):
                        plsc.store_scatter(ob, [rows, colb + d], vs[i])
                return carry
            lax.fori_loop(0, 32, sh_body, 0)

        def fire_out(c, ob, osem):
            pltpu.async_copy(tq_hbm.at[pl.ds(c * QCHUNK, QCHUNK)], ob, osem)

        def wait_out(ob, osem):
            pltpu.make_async_copy(
                ob, tq_hbm.at[pl.ds(0, QCHUNK)], osem).wait()

        n_mine = (n_full - wid + NW - 1) // NW

        fire_in(wid, in0, isem0)

        def pair(j, carry):
            c0 = wid + 2 * j * NW
            c1 = c0 + NW

            @pl.when(2 * j + 1 < n_mine)
            def _():
                fire_in(c1, in1, isem1)
            wait_in(in0, isem0)

            @pl.when(j > 0)
            def _():
                wait_out(o0, osem0)
            shuffle(in0, o0)
            pltpu.async_copy(o0, tq_hbm.at[pl.ds(c0 * QCHUNK, QCHUNK)], osem0)

            @pl.when(2 * j + 1 < n_mine)
            def _():
                c2 = c0 + 2 * NW

                @pl.when(2 * j + 2 < n_mine)
                def _():
                    fire_in(c2, in0, isem0)
                wait_in(in1, isem1)

                @pl.when(j > 0)
                def _():
                    wait_out(o1, osem1)
                shuffle(in1, o1)
                pltpu.async_copy(
                    o1, tq_hbm.at[pl.ds(c1 * QCHUNK, QCHUNK)], osem1)
            return carry

        lax.fori_loop(0, (n_mine + 1) // 2, pair, 0)

        @pl.when(n_mine > 0)
        def _():
            wait_out(o0, osem0)

        @pl.when(n_mine > 1)
        def _():
            wait_out(o1, osem1)

        if tail_v:
            @pl.when(wid == NW - 1)
            def _():
                tq0 = (n_full * VCHUNK) // 4
                tq_n = tail_v // 4
                pltpu.sync_copy(tail_hbm, o0.at[pl.ds(0, tq_n)])
                pltpu.sync_copy(o0.at[pl.ds(0, tq_n)],
                                tq_hbm.at[pl.ds(tq0, tq_n)])

    return body


def _make_gather(S, Bt):
    B = S * Bt
    b_per_w = B // NW
    n_idx_rows = b_per_w // SUB
    cpw = b_per_w // CHUNK
    n_pairs = cpw // 2
    q_per_s = Bt // CHUNK
    rows_per_chunk = CHUNK // SUB
    mesh = plsc.VectorSubcoreMesh(core_axis_name="c", subcore_axis_name="s")

    @functools.partial(
        pl.kernel,
        mesh=mesh,
        out_type=jax.ShapeDtypeStruct((S, D, Bt), jnp.float32),
        scratch_types=[
            pltpu.VMEM((n_idx_rows, SUB), jnp.int32),
            pltpu.VMEM((CHUNK, D), jnp.float32),
            pltpu.VMEM((CHUNK, D), jnp.float32),
            pltpu.VMEM((D, CHUNK), jnp.float32),
            pltpu.VMEM((D, CHUNK), jnp.float32),
            pltpu.SemaphoreType.DMA,
            pltpu.SemaphoreType.DMA,
            pltpu.SemaphoreType.DMA,
            pltpu.SemaphoreType.DMA,
        ],
        compiler_params=pltpu.CompilerParams(
            use_tc_tiling_on_sc=False, needs_layout_passes=False,
            disable_bounds_checks=True),
    )
    def body(idx_hbm, table_hbm, out_hbm, idx_v, g0, g1, t0, t1,
             gsem0, gsem1, osem0, osem1):
        wid = lax.axis_index("s") * NC + lax.axis_index("c")
        h0 = wid * cpw
        pltpu.sync_copy(idx_hbm.at[wid], idx_v)
        iota16 = lax.iota(jnp.int32, L)
        cols_d = [jnp.full((L,), d, jnp.int32) for d in range(D)]

        def fire_g(hl, g, gsem):
            for k in range(SPC):
                pltpu.async_copy(
                    table_hbm.at[idx_v.at[hl * rows_per_chunk + k]],
                    g.at[pl.ds(k * SUB, SUB)],
                    gsem,
                )

        def wait_g(g, gsem):
            pltpu.make_async_copy(
                table_hbm.at[pl.ds(0, CHUNK)], g, gsem).wait()

        def transpose(hl, g, t):
            def tr_body(gi, carry):
                rows = gi * L + iota16
                for d0 in range(0, D, 16):
                    vs = [plsc.load_gather(g, [rows, cols_d[d]])
                          for d in range(d0, d0 + 16)]
                    for i, d in enumerate(range(d0, d0 + 16)):
                        t[d, pl.ds(gi * L, L)] = vs[i]
                return carry
            lax.fori_loop(0, CHUNK // L, tr_body, 0)

        def fire_w(hl, t, osem):
            h = h0 + hl
            s = h // q_per_s
            b0 = (h % q_per_s) * CHUNK
            pltpu.async_copy(t, out_hbm.at[s, :, pl.ds(b0, CHUNK)], osem)

        def wait_w(t, osem):
            pltpu.make_async_copy(
                t, out_hbm.at[0, :, pl.ds(0, CHUNK)], osem).wait()

        fire_g(0, g0, gsem0)

        def pair(j, carry):
            hl = 2 * j
            wait_g(g0, gsem0)
            fire_g(hl + 1, g1, gsem1)

            @pl.when(j > 0)
            def _():
                wait_w(t0, osem0)
            transpose(hl, g0, t0)
            fire_w(hl, t0, osem0)

            wait_g(g1, gsem1)

            @pl.when(j < n_pairs - 1)
            def _():
                fire_g(hl + 2, g0, gsem0)

            @pl.when(j > 0)
            def _():
                wait_w(t1, osem1)
            transpose(hl + 1, g1, t1)
            fire_w(hl + 1, t1, osem1)
            return carry

        lax.fori_loop(0, n_pairs, pair, 0)
        wait_w(t0, osem0)
        wait_w(t1, osem1)

    return body


def kernel(X, table):
    Bt, S, _ = X.shape
    V = table.shape[0]
    idx3 = jnp.transpose(X, (2, 1, 0))[VAR_IDX].reshape(NW, S * Bt // (NW * SUB), SUB)
    tablet = jnp.transpose(table)
    n_tail = V % VCHUNK
    tail16 = table[V - n_tail:].reshape(n_tail // 4, 128)
    tableq = _make_pack(V)(tablet, tail16)
    out3 = _make_gather(S, Bt)(idx3, tableq.reshape(V, D))
    return jnp.transpose(out3, (2, 0, 1))
